# baseline (speedup 1.0000x reference)
import functools

import jax
import jax.numpy as jnp
from jax import lax
from jax.experimental import pallas as pl
from jax.experimental.pallas import tpu as pltpu
from jax.experimental.pallas import tpu_sc as plsc

RADIUS = 0.09
KS = 3
GRID = 10
TQ = 128
TP = 128
TOT_STEPS = 896
EPS = 1e-8


def _cconv_body(qb_ref, pb_ref, first_ref, last_ref, valid_ref,
                qp_ref, pp_ref, f_ref, wm_ref, b_ref,
                out_ref, s_acc, cnt_acc):
    i = pl.program_id(0)

    @pl.when(first_ref[i] == 1)
    def _init():
        s_acc[...] = jnp.zeros_like(s_acc)
        cnt_acc[...] = jnp.zeros_like(cnt_acc)

    def _accum_pairs(rx, ry, rz, r2, mask):
        norm = jnp.sqrt(jnp.maximum(r2, EPS))
        ninf = jnp.maximum(
            jnp.maximum(jnp.abs(rx), jnp.abs(ry)),
            jnp.maximum(jnp.abs(rz), EPS))
        scale = jnp.where(r2 > EPS, norm / ninf, 0.0)
        gx = rx * scale + 1.0
        gy = ry * scale + 1.0
        gz = rz * scale + 1.0
        wx = [jnp.maximum(1.0 - jnp.abs(gx - c), 0.0) for c in (0.0, 1.0, 2.0)]
        wy = [jnp.maximum(1.0 - jnp.abs(gy - c), 0.0) for c in (0.0, 1.0, 2.0)]
        wz = [jnp.maximum(1.0 - jnp.abs(gz - c), 0.0) for c in (0.0, 1.0, 2.0)]
        coeffs = []
        for cx in range(KS):
            wxm = wx[cx] * mask
            for cy in range(KS):
                wxy = wxm * wy[cy]
                for cz in range(KS):
                    coeffs.append(wxy * wz[cz])
        a = jnp.concatenate(coeffs, axis=0)
        s_acc[...] += jax.lax.dot_general(
            a, f_ref[...], (((1,), (0,)), ((), ())),
            preferred_element_type=jnp.float32)
        cnt_acc[...] += jnp.sum(mask, axis=1, keepdims=True)

    @pl.when(valid_ref[i] == 1)
    def _accum():
        inv_r = 1.0 / RADIUS
        rx = (pp_ref[0:1, :] - qp_ref[:, 0:1]) * inv_r
        ry = (pp_ref[1:2, :] - qp_ref[:, 1:2]) * inv_r
        rz = (pp_ref[2:3, :] - qp_ref[:, 2:3]) * inv_r
        r2 = rx * rx + ry * ry + rz * rz
        mask = (r2 <= 1.0).astype(jnp.float32)
        _accum_pairs(rx, ry, rz, r2, mask)

    @pl.when(last_ref[i] == 1)
    def _finish():
        s3 = s_acc[...].reshape(KS * KS * KS, TQ, -1)
        per_cell = jax.lax.dot_general(
            s3, wm_ref[...], (((2,), (1,)), ((0,), (0,))),
            preferred_element_type=jnp.float32)
        o = jnp.sum(per_cell, axis=0)
        n = jnp.maximum(cnt_acc[...], 1.0)
        out_ref[...] = o / n + b_ref[0:1, :]


_SC_CHUNK = 128


def _sc_permute_gather(table, idx, npass):
    info = plsc.get_sparse_core_info()
    nc, ns = info.num_cores, info.num_subcores
    nw = nc * ns
    b_tot = idx.shape[0]
    assert b_tot == nw * npass * _SC_CHUNK
    d = table.shape[1]
    mesh = plsc.VectorSubcoreMesh(core_axis_name="c", subcore_axis_name="s")

    bpw = npass * _SC_CHUNK

    @functools.partial(
        pl.kernel, mesh=mesh,
        compiler_params=pltpu.CompilerParams(use_tc_tiling_on_sc=False),
        out_type=jax.ShapeDtypeStruct((b_tot, d), jnp.float32),
        scratch_types=[pltpu.VMEM((8, _SC_CHUNK), jnp.int32),
                       pltpu.VMEM((bpw, d), jnp.float32),
                       pltpu.SemaphoreType.DMA],
    )
    def k(t_hbm, idx_hbm, o_hbm, idx_v, r_v, sem):
        wid = lax.axis_index("s") * nc + lax.axis_index("c")
        base = wid * bpw
        pltpu.sync_copy(idx_hbm.at[pl.ds(wid * 8, 8)], idx_v)
        copies = []
        for p in range(npass):
            dst = pl.ds(p * _SC_CHUNK, _SC_CHUNK)
            copies.append(
                pltpu.async_copy(t_hbm.at[idx_v.at[p]], r_v.at[dst], sem))
        for c in copies:
            c.wait()
        pltpu.sync_copy(r_v, o_hbm.at[pl.ds(base, bpw)])

    idx2 = jnp.pad(idx.reshape(nw, npass, _SC_CHUNK),
                   ((0, 0), (0, 8 - npass), (0, 0)),
                   constant_values=0).reshape(nw * 8, _SC_CHUNK)
    return k(table, idx2)


def _sc_scatter_rows(rows, idx, npass):
    info = plsc.get_sparse_core_info()
    nc, ns = info.num_cores, info.num_subcores
    nw = nc * ns
    b_tot = rows.shape[0]
    assert b_tot == nw * npass * _SC_CHUNK
    d = rows.shape[1]
    mesh = plsc.VectorSubcoreMesh(core_axis_name="c", subcore_axis_name="s")

    bpw = npass * _SC_CHUNK

    @functools.partial(
        pl.kernel, mesh=mesh,
        compiler_params=pltpu.CompilerParams(use_tc_tiling_on_sc=False),
        out_type=jax.ShapeDtypeStruct((b_tot, d), jnp.float32),
        scratch_types=[pltpu.VMEM((8, _SC_CHUNK), jnp.int32),
                       pltpu.VMEM((bpw, d), jnp.float32),
                       pltpu.SemaphoreType.DMA],
    )
    def k(rows_hbm, idx_hbm, out_hbm, idx_v, r_v, sem):
        wid = lax.axis_index("s") * nc + lax.axis_index("c")
        base = wid * bpw
        pltpu.sync_copy(idx_hbm.at[pl.ds(wid * 8, 8)], idx_v)
        pltpu.sync_copy(rows_hbm.at[pl.ds(base, bpw)], r_v)
        copies = []
        for p in range(npass):
            src = pl.ds(p * _SC_CHUNK, _SC_CHUNK)
            copies.append(
                pltpu.async_copy(r_v.at[src], out_hbm.at[idx_v.at[p]], sem))
        for c in copies:
            c.wait()

    idx2 = jnp.pad(idx.reshape(nw, npass, _SC_CHUNK),
                   ((0, 0), (0, 8 - npass), (0, 0)),
                   constant_values=0).reshape(nw * 8, _SC_CHUNK)
    return k(rows, idx2)


def kernel(feats, points, W, b):
    n, c_in = feats.shape
    c_out = W.shape[-1]
    qb = (n + TQ - 1) // TQ
    np_pad = qb * TQ
    ncell = GRID * GRID * GRID

    feats = feats.astype(jnp.float32)
    points = points.astype(jnp.float32)

    ijk = jnp.clip((points * GRID).astype(jnp.int32), 0, GRID - 1)
    cell = (ijk[:, 0] * GRID + ijk[:, 1]) * GRID + ijk[:, 2]
    order = jnp.argsort(cell).astype(jnp.int32)

    nw_rows = 32 * _SC_CHUNK
    npass = -(-max(np_pad, n) // nw_rows)
    b_tot = npass * nw_rows
    table_fp = jnp.concatenate([
        jnp.concatenate([feats, jnp.pad(points, ((0, 0), (0, 13)))], axis=1),
        jnp.concatenate([jnp.zeros((1, c_in), jnp.float32),
                         jnp.full((1, 16), 1e6, jnp.float32)], axis=1),
    ], axis=0)
    idx_g = jnp.concatenate(
        [order, jnp.full((b_tot - n,), n, jnp.int32)])
    rows_fp = _sc_permute_gather(table_fp, idx_g, npass)
    feats_pad = rows_fp[:np_pad, :c_in]
    pts8 = rows_fp[:np_pad, c_in:c_in + 8]
    ptsT = rows_fp[:np_pad, c_in:c_in + 3].T
    pts_s = rows_fp[:n, c_in:c_in + 3]
    ijk_s = jnp.clip((pts_s * GRID).astype(jnp.int32), 0, GRID - 1)
    cell_s = (ijk_s[:, 0] * GRID + ijk_s[:, 1]) * GRID + ijk_s[:, 2]

    blk = jnp.arange(n, dtype=jnp.int32) // TQ
    memb = jnp.zeros((qb, ncell), jnp.float32).at[blk, cell_s].set(1.0)
    cid = jnp.arange(ncell, dtype=jnp.int32)
    cx, cy, cz = cid // (GRID * GRID), (cid // GRID) % GRID, cid % GRID
    nbmat = ((jnp.abs(cx[:, None] - cx[None, :]) <= 1)
             & (jnp.abs(cy[:, None] - cy[None, :]) <= 1)
             & (jnp.abs(cz[:, None] - cz[None, :]) <= 1)).astype(jnp.float32)
    cellcov = (memb @ nbmat > 0).astype(jnp.float32)
    cov = cellcov @ memb.T > 0
    counts = jnp.sum(cov, axis=1).astype(jnp.int32)
    pb_sorted = jnp.argsort(~cov, axis=1, stable=True).astype(jnp.int32)
    ends = jnp.cumsum(counts)
    starts = ends - counts
    total = ends[-1]

    tot = min(TOT_STEPS, qb * qb)
    i_flat = jnp.arange(tot, dtype=jnp.int32)
    valid_tab = (i_flat < total).astype(jnp.int32)
    i_cl = jnp.minimum(i_flat, total - 1)
    r_of_i = jnp.searchsorted(ends, i_cl, side='right').astype(jnp.int32)
    j_of_i = i_cl - starts[r_of_i]
    qb_tab = r_of_i
    pb_tab = pb_sorted[r_of_i, j_of_i]
    first_tab = ((j_of_i == 0) & (valid_tab == 1)).astype(jnp.int32)
    last_tab = ((j_of_i == counts[r_of_i] - 1)
                & (valid_tab == 1)).astype(jnp.int32)

    wm = W.astype(jnp.float32).reshape(KS * KS * KS, c_in, c_out)
    b2 = b.astype(jnp.float32).reshape(1, c_out)

    grid_spec = pltpu.PrefetchScalarGridSpec(
        num_scalar_prefetch=5,
        grid=(tot,),
        in_specs=[
            pl.BlockSpec((TQ, 8), lambda i, qt, pt, ft, lt, vt: (qt[i], 0)),
            pl.BlockSpec((3, TP), lambda i, qt, pt, ft, lt, vt: (0, pt[i])),
            pl.BlockSpec((TP, c_in),
                         lambda i, qt, pt, ft, lt, vt: (pt[i], 0)),
            pl.BlockSpec((KS * KS * KS, c_in, c_out),
                         lambda i, qt, pt, ft, lt, vt: (0, 0, 0)),
            pl.BlockSpec((1, c_out), lambda i, qt, pt, ft, lt, vt: (0, 0)),
        ],
        out_specs=pl.BlockSpec((TQ, c_out),
                               lambda i, qt, pt, ft, lt, vt: (qt[i], 0)),
        scratch_shapes=[
            pltpu.VMEM((KS * KS * KS * TQ, c_in), jnp.float32),
            pltpu.VMEM((TQ, 1), jnp.float32),
        ],
    )
    out_sorted = pl.pallas_call(
        _cconv_body,
        grid_spec=grid_spec,
        out_shape=jax.ShapeDtypeStruct((np_pad, c_out), jnp.float32),
    )(qb_tab, pb_tab, first_tab, last_tab, valid_tab,
      pts8, ptsT, feats_pad, wm, b2)

    rows_o = jnp.pad(
        jnp.concatenate(
            [out_sorted, jnp.zeros((b_tot - np_pad, c_out), jnp.float32)],
            axis=0),
        ((0, 0), (0, 128 - c_out)))
    idx_s = jnp.concatenate(
        [order, jnp.arange(n, b_tot, dtype=jnp.int32)])
    out = _sc_scatter_rows(rows_o, idx_s, npass)
    return out[:n, :c_out]


# speedup vs baseline: 1.0148x; 1.0148x over previous
import functools

import jax
import jax.numpy as jnp
from jax import lax
from jax.experimental import pallas as pl
from jax.experimental.pallas import tpu as pltpu
from jax.experimental.pallas import tpu_sc as plsc

RADIUS = 0.09
KS = 3
GRID = 10
TQ = 128
TP = 128
TOT_STEPS = 896
EPS = 1e-8


def _cconv_body(qb_ref, pb_ref, first_ref, last_ref, valid_ref,
                qp_ref, pp_ref, f_ref, wm_ref, b_ref,
                out_ref, s_acc, cnt_acc):
    i = pl.program_id(0)

    @pl.when(first_ref[i] == 1)
    def _init():
        s_acc[...] = jnp.zeros_like(s_acc)
        cnt_acc[...] = jnp.zeros_like(cnt_acc)

    def _accum_pairs(rx, ry, rz, r2, mask):
        norm = jnp.sqrt(jnp.maximum(r2, EPS))
        ninf = jnp.maximum(
            jnp.maximum(jnp.abs(rx), jnp.abs(ry)),
            jnp.maximum(jnp.abs(rz), EPS))
        scale = jnp.where(r2 > EPS, norm / ninf, 0.0)
        gx = rx * scale + 1.0
        gy = ry * scale + 1.0
        gz = rz * scale + 1.0
        wx = [jnp.maximum(1.0 - jnp.abs(gx - c), 0.0) for c in (0.0, 1.0, 2.0)]
        wy = [jnp.maximum(1.0 - jnp.abs(gy - c), 0.0) for c in (0.0, 1.0, 2.0)]
        wz = [jnp.maximum(1.0 - jnp.abs(gz - c), 0.0) for c in (0.0, 1.0, 2.0)]
        coeffs = []
        for cx in range(KS):
            wxm = wx[cx] * mask
            for cy in range(KS):
                wxy = wxm * wy[cy]
                for cz in range(KS):
                    coeffs.append(wxy * wz[cz])
        a = jnp.concatenate(coeffs, axis=0)
        s_acc[...] += jax.lax.dot_general(
            a, f_ref[...], (((1,), (0,)), ((), ())),
            preferred_element_type=jnp.float32)
        cnt_acc[...] += jnp.sum(mask, axis=1, keepdims=True)

    @pl.when(valid_ref[i] == 1)
    def _accum():
        inv_r = 1.0 / RADIUS
        rx = (pp_ref[0:1, :] - qp_ref[:, 0:1]) * inv_r
        ry = (pp_ref[1:2, :] - qp_ref[:, 1:2]) * inv_r
        rz = (pp_ref[2:3, :] - qp_ref[:, 2:3]) * inv_r
        r2 = rx * rx + ry * ry + rz * rz
        mask = (r2 <= 1.0).astype(jnp.float32)
        _accum_pairs(rx, ry, rz, r2, mask)

    @pl.when(last_ref[i] == 1)
    def _finish():
        s3 = s_acc[...].reshape(KS * KS * KS, TQ, -1)
        per_cell = jax.lax.dot_general(
            s3, wm_ref[...], (((2,), (1,)), ((0,), (0,))),
            preferred_element_type=jnp.float32)
        o = jnp.sum(per_cell, axis=0)
        n = jnp.maximum(cnt_acc[...], 1.0)
        out_ref[...] = o / n + b_ref[0:1, :]


_SC_CHUNK = 128


def _sc_permute_gather(table_f, table_p, idx, npass):
    info = plsc.get_sparse_core_info()
    nc, ns = info.num_cores, info.num_subcores
    nw = nc * ns
    b_tot = idx.shape[0]
    assert b_tot == nw * npass * _SC_CHUNK
    d_f = table_f.shape[1]
    d_p = table_p.shape[1]
    mesh = plsc.VectorSubcoreMesh(core_axis_name="c", subcore_axis_name="s")

    bpw = npass * _SC_CHUNK

    @functools.partial(
        pl.kernel, mesh=mesh,
        compiler_params=pltpu.CompilerParams(use_tc_tiling_on_sc=False),
        out_type=[jax.ShapeDtypeStruct((b_tot, d_f), jnp.float32),
                  jax.ShapeDtypeStruct((b_tot, d_p), jnp.float32)],
        scratch_types=[pltpu.VMEM((8, _SC_CHUNK), jnp.int32),
                       pltpu.VMEM((bpw, d_f), jnp.float32),
                       pltpu.VMEM((bpw, d_p), jnp.float32),
                       pltpu.SemaphoreType.DMA],
    )
    def k(tf_hbm, tp_hbm, idx_hbm, of_hbm, op_hbm, idx_v, rf_v, rp_v, sem):
        wid = lax.axis_index("s") * nc + lax.axis_index("c")
        base = wid * bpw
        pltpu.sync_copy(idx_hbm.at[pl.ds(wid * 8, 8)], idx_v)
        copies = []
        for p in range(npass):
            dst = pl.ds(p * _SC_CHUNK, _SC_CHUNK)
            copies.append(
                pltpu.async_copy(tf_hbm.at[idx_v.at[p]], rf_v.at[dst], sem))
            copies.append(
                pltpu.async_copy(tp_hbm.at[idx_v.at[p]], rp_v.at[dst], sem))
        for c in copies:
            c.wait()
        pltpu.sync_copy(rf_v, of_hbm.at[pl.ds(base, bpw)])
        pltpu.sync_copy(rp_v, op_hbm.at[pl.ds(base, bpw)])

    idx2 = jnp.pad(idx.reshape(nw, npass, _SC_CHUNK),
                   ((0, 0), (0, 8 - npass), (0, 0)),
                   constant_values=0).reshape(nw * 8, _SC_CHUNK)
    return k(table_f, table_p, idx2)


def _sc_scatter_rows(rows, idx, npass):
    info = plsc.get_sparse_core_info()
    nc, ns = info.num_cores, info.num_subcores
    nw = nc * ns
    b_tot = rows.shape[0]
    assert b_tot == nw * npass * _SC_CHUNK
    d = rows.shape[1]
    mesh = plsc.VectorSubcoreMesh(core_axis_name="c", subcore_axis_name="s")

    bpw = npass * _SC_CHUNK

    @functools.partial(
        pl.kernel, mesh=mesh,
        compiler_params=pltpu.CompilerParams(use_tc_tiling_on_sc=False),
        out_type=jax.ShapeDtypeStruct((b_tot, d), jnp.float32),
        scratch_types=[pltpu.VMEM((8, _SC_CHUNK), jnp.int32),
                       pltpu.VMEM((bpw, d), jnp.float32),
                       pltpu.SemaphoreType.DMA],
    )
    def k(rows_hbm, idx_hbm, out_hbm, idx_v, r_v, sem):
        wid = lax.axis_index("s") * nc + lax.axis_index("c")
        base = wid * bpw
        pltpu.sync_copy(idx_hbm.at[pl.ds(wid * 8, 8)], idx_v)
        pltpu.sync_copy(rows_hbm.at[pl.ds(base, bpw)], r_v)
        copies = []
        for p in range(npass):
            src = pl.ds(p * _SC_CHUNK, _SC_CHUNK)
            copies.append(
                pltpu.async_copy(r_v.at[src], out_hbm.at[idx_v.at[p]], sem))
        for c in copies:
            c.wait()

    idx2 = jnp.pad(idx.reshape(nw, npass, _SC_CHUNK),
                   ((0, 0), (0, 8 - npass), (0, 0)),
                   constant_values=0).reshape(nw * 8, _SC_CHUNK)
    return k(rows, idx2)


def kernel(feats, points, W, b):
    n, c_in = feats.shape
    c_out = W.shape[-1]
    qb = (n + TQ - 1) // TQ
    np_pad = qb * TQ
    ncell = GRID * GRID * GRID

    feats = feats.astype(jnp.float32)
    points = points.astype(jnp.float32)

    ijk = jnp.clip((points * GRID).astype(jnp.int32), 0, GRID - 1)
    cell = (ijk[:, 0] * GRID + ijk[:, 1]) * GRID + ijk[:, 2]
    order = jnp.argsort(cell).astype(jnp.int32)

    nw_rows = 32 * _SC_CHUNK
    npass = -(-max(np_pad, n) // nw_rows)
    b_tot = npass * nw_rows
    table_p = jnp.concatenate(
        [jnp.pad(points, ((0, 0), (0, 125)), constant_values=0.0),
         jnp.full((1, 128), 1e6, jnp.float32)], axis=0)
    table_f = jnp.concatenate(
        [feats, jnp.zeros((1, c_in), jnp.float32)], axis=0)
    idx_g = jnp.concatenate(
        [order, jnp.full((b_tot - n,), n, jnp.int32)])
    rows_f, rows_p = _sc_permute_gather(table_f, table_p, idx_g, npass)
    feats_pad = rows_f[:np_pad]
    pts8 = rows_p[:np_pad, :8]
    ptsT = rows_p[:np_pad, :3].T
    pts_s = rows_p[:n, :3]
    ijk_s = jnp.clip((pts_s * GRID).astype(jnp.int32), 0, GRID - 1)
    cell_s = (ijk_s[:, 0] * GRID + ijk_s[:, 1]) * GRID + ijk_s[:, 2]

    blk = jnp.arange(n, dtype=jnp.int32) // TQ
    memb = jnp.zeros((qb, ncell), jnp.float32).at[blk, cell_s].set(1.0)
    cid = jnp.arange(ncell, dtype=jnp.int32)
    cx, cy, cz = cid // (GRID * GRID), (cid // GRID) % GRID, cid % GRID
    nbmat = ((jnp.abs(cx[:, None] - cx[None, :]) <= 1)
             & (jnp.abs(cy[:, None] - cy[None, :]) <= 1)
             & (jnp.abs(cz[:, None] - cz[None, :]) <= 1)).astype(jnp.float32)
    cellcov = (memb @ nbmat > 0).astype(jnp.float32)
    cov = cellcov @ memb.T > 0
    counts = jnp.sum(cov, axis=1).astype(jnp.int32)
    pb_sorted = jnp.argsort(~cov, axis=1, stable=True).astype(jnp.int32)
    ends = jnp.cumsum(counts)
    starts = ends - counts
    total = ends[-1]

    tot = min(TOT_STEPS, qb * qb)
    i_flat = jnp.arange(tot, dtype=jnp.int32)
    valid_tab = (i_flat < total).astype(jnp.int32)
    i_cl = jnp.minimum(i_flat, total - 1)
    r_of_i = jnp.searchsorted(ends, i_cl, side='right').astype(jnp.int32)
    j_of_i = i_cl - starts[r_of_i]
    qb_tab = r_of_i
    pb_tab = pb_sorted[r_of_i, j_of_i]
    first_tab = ((j_of_i == 0) & (valid_tab == 1)).astype(jnp.int32)
    last_tab = ((j_of_i == counts[r_of_i] - 1)
                & (valid_tab == 1)).astype(jnp.int32)

    wm = W.astype(jnp.float32).reshape(KS * KS * KS, c_in, c_out)
    b2 = b.astype(jnp.float32).reshape(1, c_out)

    grid_spec = pltpu.PrefetchScalarGridSpec(
        num_scalar_prefetch=5,
        grid=(tot,),
        in_specs=[
            pl.BlockSpec((TQ, 8), lambda i, qt, pt, ft, lt, vt: (qt[i], 0)),
            pl.BlockSpec((3, TP), lambda i, qt, pt, ft, lt, vt: (0, pt[i])),
            pl.BlockSpec((TP, c_in),
                         lambda i, qt, pt, ft, lt, vt: (pt[i], 0)),
            pl.BlockSpec((KS * KS * KS, c_in, c_out),
                         lambda i, qt, pt, ft, lt, vt: (0, 0, 0)),
            pl.BlockSpec((1, c_out), lambda i, qt, pt, ft, lt, vt: (0, 0)),
        ],
        out_specs=pl.BlockSpec((TQ, c_out),
                               lambda i, qt, pt, ft, lt, vt: (qt[i], 0)),
        scratch_shapes=[
            pltpu.VMEM((KS * KS * KS * TQ, c_in), jnp.float32),
            pltpu.VMEM((TQ, 1), jnp.float32),
        ],
    )
    out_sorted = pl.pallas_call(
        _cconv_body,
        grid_spec=grid_spec,
        out_shape=jax.ShapeDtypeStruct((np_pad, c_out), jnp.float32),
    )(qb_tab, pb_tab, first_tab, last_tab, valid_tab,
      pts8, ptsT, feats_pad, wm, b2)

    rows_o = jnp.pad(
        jnp.concatenate(
            [out_sorted, jnp.zeros((b_tot - np_pad, c_out), jnp.float32)],
            axis=0),
        ((0, 0), (0, 128 - c_out)))
    idx_s = jnp.concatenate(
        [order, jnp.arange(n, b_tot, dtype=jnp.int32)])
    out = _sc_scatter_rows(rows_o, idx_s, npass)
    return out[:n, :c_out]
